# Initial kernel scaffold; baseline (speedup 1.0000x reference)
import functools

import jax
import jax.numpy as jnp
from jax.experimental import pallas as pl
from jax.experimental.pallas import tpu as pltpu


def _attn_kernel(rb_size, g_ref, r_ref, gl_ref, rl_ref, v_ref, out_ref, kn_ref):
    rb = pl.program_id(1)

    @pl.when(rb == 0)
    def _prep():
        R = r_ref[0]
        rm = (rl_ref[0] != 0).astype(jnp.float32)
        nr = jnp.sum(rm)
        r_mean = jnp.sum(R * rm, axis=1, keepdims=True) / jnp.maximum(nr, 1.0)
        r_bar = (R - r_mean) * rm
        r_n2 = jnp.sum(r_bar * r_bar, axis=0, keepdims=True)
        kn_ref[...] = r_bar * jax.lax.rsqrt(jnp.where(r_n2 == 0.0, 1.0, r_n2))

    G = g_ref[0]
    gm = (gl_ref[0] != 0).astype(jnp.float32)
    ng = jnp.sum(gm)
    g_mean = jnp.sum(G * gm, axis=1, keepdims=True) / jnp.maximum(ng, 1.0)
    sl = pl.ds(rb * rb_size, rb_size)
    gb = (G[:, sl] - g_mean) * gm[:, sl]
    g_n2 = jnp.sum(gb * gb, axis=0, keepdims=True)
    qn = gb * jax.lax.rsqrt(jnp.where(g_n2 == 0.0, 1.0, g_n2))

    logits = jax.lax.dot_general(
        qn, kn_ref[...], (((0,), (0,)), ((), ())),
        preferred_element_type=jnp.float32)
    rmb = rl_ref[0] != 0
    e = jnp.where(rmb, jnp.exp(logits), 0.0)
    s = jnp.maximum(jnp.sum(e, axis=1, keepdims=True), 1e-30)
    V = v_ref[0]
    o0 = jnp.sum(e * V[0:1, :], axis=1, keepdims=True)
    o1 = jnp.sum(e * V[1:2, :], axis=1, keepdims=True)
    o2 = jnp.sum(e * V[2:3, :], axis=1, keepdims=True)
    out_ref[0] = jnp.concatenate([o0, o1, o2], axis=1) / s


def _merge_kernel(n_ch, col_ref, gl_ref, rl_ref, out_ref):
    hw = out_ref.shape[2]
    acc = jnp.full((3, hw), -1.0, jnp.float32)
    for c in range(1, n_ch):
        gm = gl_ref[c] != 0
        rm = rl_ref[c] != 0
        ng = jnp.sum(gm.astype(jnp.float32))
        nr = jnp.sum(rm.astype(jnp.float32))
        valid = (ng > 1.0) & (nr > 1.0)
        sel = valid & gm
        acc = jnp.where(sel, col_ref[c - 1].T, acc)
    out_ref[0] = acc


def kernel(gray_feature, rgb_feature, rgb_image, gray_label, rgb_label):
    b, c, h, w = gray_feature.shape
    n_ch = gray_label.shape[1]
    hw = h * w
    n_cls = n_ch - 1
    n_pairs = b * n_cls
    rb_size = min(256, hw)
    n_rb = hw // rb_size

    gf = gray_feature.reshape(b, c, hw)
    rf = rgb_feature.reshape(b, c, hw)
    vi = rgb_image.reshape(b, 3, hw)
    gl = gray_label.reshape(b * n_ch, 1, hw)
    rl = rgb_label.reshape(b * n_ch, 1, hw)

    def lab_idx(p, r):
        return ((p // n_cls) * n_ch + p % n_cls + 1, 0, 0)

    colorized = pl.pallas_call(
        functools.partial(_attn_kernel, rb_size),
        grid=(n_pairs, n_rb),
        in_specs=[
            pl.BlockSpec((1, c, hw), lambda p, r: (p // n_cls, 0, 0)),
            pl.BlockSpec((1, c, hw), lambda p, r: (p // n_cls, 0, 0)),
            pl.BlockSpec((1, 1, hw), lab_idx),
            pl.BlockSpec((1, 1, hw), lab_idx),
            pl.BlockSpec((1, 3, hw), lambda p, r: (p // n_cls, 0, 0)),
        ],
        out_specs=pl.BlockSpec((1, rb_size, 3), lambda p, r: (p, r, 0)),
        out_shape=jax.ShapeDtypeStruct((n_pairs, hw, 3), jnp.float32),
        scratch_shapes=[pltpu.VMEM((c, hw), jnp.float32)],
    )(gf, rf, gl, rl, vi)

    canvas = pl.pallas_call(
        functools.partial(_merge_kernel, n_ch),
        grid=(b,),
        in_specs=[
            pl.BlockSpec((n_cls, hw, 3), lambda i: (i, 0, 0)),
            pl.BlockSpec((n_ch, 1, hw), lambda i: (i, 0, 0)),
            pl.BlockSpec((n_ch, 1, hw), lambda i: (i, 0, 0)),
        ],
        out_specs=pl.BlockSpec((1, 3, hw), lambda i: (i, 0, 0)),
        out_shape=jax.ShapeDtypeStruct((b, 3, hw), jnp.float32),
    )(colorized, gl, rl)
    return canvas.reshape(b, 3, h, w)


# speedup vs baseline: 2.0475x; 2.0475x over previous
import functools

import jax
import jax.numpy as jnp
from jax.experimental import pallas as pl
from jax.experimental.pallas import tpu as pltpu


def _attn_kernel(rb_size, g_ref, r_ref, gl_ref, rl_ref, v_ref, out_ref, kn_ref):
    rb = pl.program_id(1)

    @pl.when(rb == 0)
    def _prep():
        R = r_ref[0]
        rm = (rl_ref[0] != 0).astype(jnp.float32)
        nr = jnp.sum(rm)
        r_mean = jnp.sum(R * rm, axis=1, keepdims=True) / jnp.maximum(nr, 1.0)
        r_bar = (R - r_mean) * rm
        r_n2 = jnp.sum(r_bar * r_bar, axis=0, keepdims=True)
        kn_ref[...] = r_bar * jax.lax.rsqrt(jnp.where(r_n2 == 0.0, 1.0, r_n2))

    G = g_ref[0]
    gm = (gl_ref[0] != 0).astype(jnp.float32)
    ng = jnp.sum(gm)
    g_mean = jnp.sum(G * gm, axis=1, keepdims=True) / jnp.maximum(ng, 1.0)
    sl = pl.ds(rb * rb_size, rb_size)
    gb = (g_ref[0, :, sl] - g_mean) * (gl_ref[0, :, sl] != 0).astype(jnp.float32)
    g_n2 = jnp.sum(gb * gb, axis=0, keepdims=True)
    qn = gb * jax.lax.rsqrt(jnp.where(g_n2 == 0.0, 1.0, g_n2))

    logits = jax.lax.dot_general(
        qn, kn_ref[...], (((0,), (0,)), ((), ())),
        preferred_element_type=jnp.float32)
    rmb = rl_ref[0] != 0
    e = jnp.where(rmb, jnp.exp(logits), 0.0)
    s = jnp.maximum(jnp.sum(e, axis=1, keepdims=True), 1e-30)
    V = v_ref[0]
    o0 = jnp.sum(e * V[0:1, :], axis=1, keepdims=True)
    o1 = jnp.sum(e * V[1:2, :], axis=1, keepdims=True)
    o2 = jnp.sum(e * V[2:3, :], axis=1, keepdims=True)
    out_ref[0] = jnp.concatenate([o0, o1, o2], axis=1) / s


def _merge_kernel(n_ch, col_ref, gl_ref, rl_ref, out_ref):
    hw = out_ref.shape[2]
    acc = jnp.full((3, hw), -1.0, jnp.float32)
    for c in range(1, n_ch):
        gm = gl_ref[c] != 0
        rm = rl_ref[c] != 0
        ng = jnp.sum(gm.astype(jnp.float32))
        nr = jnp.sum(rm.astype(jnp.float32))
        valid = (ng > 1.0) & (nr > 1.0)
        sel = valid & gm
        acc = jnp.where(sel, col_ref[c - 1].T, acc)
    out_ref[0] = acc


def kernel(gray_feature, rgb_feature, rgb_image, gray_label, rgb_label):
    b, c, h, w = gray_feature.shape
    n_ch = gray_label.shape[1]
    hw = h * w
    n_cls = n_ch - 1
    n_pairs = b * n_cls
    rb_size = min(256, hw)
    n_rb = hw // rb_size

    gf = gray_feature.reshape(b, c, hw)
    rf = rgb_feature.reshape(b, c, hw)
    vi = rgb_image.reshape(b, 3, hw)
    gl = gray_label.reshape(b * n_ch, 1, hw)
    rl = rgb_label.reshape(b * n_ch, 1, hw)

    def lab_idx(p, r):
        return ((p // n_cls) * n_ch + p % n_cls + 1, 0, 0)

    colorized = pl.pallas_call(
        functools.partial(_attn_kernel, rb_size),
        grid=(n_pairs, n_rb),
        in_specs=[
            pl.BlockSpec((1, c, hw), lambda p, r: (p // n_cls, 0, 0)),
            pl.BlockSpec((1, c, hw), lambda p, r: (p // n_cls, 0, 0)),
            pl.BlockSpec((1, 1, hw), lab_idx),
            pl.BlockSpec((1, 1, hw), lab_idx),
            pl.BlockSpec((1, 3, hw), lambda p, r: (p // n_cls, 0, 0)),
        ],
        out_specs=pl.BlockSpec((1, rb_size, 3), lambda p, r: (p, r, 0)),
        out_shape=jax.ShapeDtypeStruct((n_pairs, hw, 3), jnp.float32),
        scratch_shapes=[pltpu.VMEM((c, hw), jnp.float32)],
    )(gf, rf, gl, rl, vi)

    canvas = pl.pallas_call(
        functools.partial(_merge_kernel, n_ch),
        grid=(b,),
        in_specs=[
            pl.BlockSpec((n_cls, hw, 3), lambda i: (i, 0, 0)),
            pl.BlockSpec((n_ch, 1, hw), lambda i: (i, 0, 0)),
            pl.BlockSpec((n_ch, 1, hw), lambda i: (i, 0, 0)),
        ],
        out_specs=pl.BlockSpec((1, 3, hw), lambda i: (i, 0, 0)),
        out_shape=jax.ShapeDtypeStruct((b, 3, hw), jnp.float32),
    )(colorized, gl, rl)
    return canvas.reshape(b, 3, h, w)


# speedup vs baseline: 2.9196x; 1.4259x over previous
import functools

import jax
import jax.numpy as jnp
from jax.experimental import pallas as pl
from jax.experimental.pallas import tpu as pltpu


def _normalize(F, mask):
    n = jnp.sum(mask)
    mean = jnp.sum(F * mask, axis=1, keepdims=True) / jnp.maximum(n, 1.0)
    bar = (F - mean) * mask
    n2 = jnp.sum(bar * bar, axis=0, keepdims=True)
    return bar * jax.lax.rsqrt(jnp.where(n2 == 0.0, 1.0, n2))


def _attn_kernel(rb_size, g_ref, r_ref, gl_ref, rl_ref, v_ref, out_ref,
                 qn_ref, kn_ref, w_ref):
    rb = pl.program_id(1)

    @pl.when(rb == 0)
    def _prep():
        gm = (gl_ref[0] != 0).astype(jnp.float32)
        rm = (rl_ref[0] != 0).astype(jnp.float32)
        qn_ref[...] = _normalize(g_ref[0], gm).astype(jnp.bfloat16)
        kn_ref[...] = _normalize(r_ref[0], rm).astype(jnp.bfloat16)
        V = v_ref[0]
        ones = jnp.ones_like(rm)
        w_ref[...] = (jnp.concatenate([V, ones], axis=0) * rm
                      ).astype(jnp.bfloat16)

    sl = pl.ds(rb * rb_size, rb_size)
    logits = jax.lax.dot_general(
        qn_ref[:, sl], kn_ref[...], (((0,), (0,)), ((), ())),
        preferred_element_type=jnp.float32)
    e = jnp.exp(logits).astype(jnp.bfloat16)
    outs = jax.lax.dot_general(
        e, w_ref[...], (((1,), (1,)), ((), ())),
        preferred_element_type=jnp.float32)
    out_ref[0] = outs[:, 0:3] / jnp.maximum(outs[:, 3:4], 1e-30)


def _merge_kernel(n_ch, col_ref, gl_ref, rl_ref, out_ref):
    hw = out_ref.shape[2]
    acc = jnp.full((3, hw), -1.0, jnp.float32)
    for c in range(1, n_ch):
        gm = gl_ref[c] != 0
        rm = rl_ref[c] != 0
        ng = jnp.sum(gm.astype(jnp.float32))
        nr = jnp.sum(rm.astype(jnp.float32))
        valid = (ng > 1.0) & (nr > 1.0)
        sel = valid & gm
        acc = jnp.where(sel, col_ref[c - 1].T, acc)
    out_ref[0] = acc


def kernel(gray_feature, rgb_feature, rgb_image, gray_label, rgb_label):
    b, c, h, w = gray_feature.shape
    n_ch = gray_label.shape[1]
    hw = h * w
    n_cls = n_ch - 1
    n_pairs = b * n_cls
    rb_size = min(256, hw)
    n_rb = hw // rb_size

    gf = gray_feature.reshape(b, c, hw)
    rf = rgb_feature.reshape(b, c, hw)
    vi = rgb_image.reshape(b, 3, hw)
    gl = gray_label.reshape(b * n_ch, 1, hw)
    rl = rgb_label.reshape(b * n_ch, 1, hw)

    def lab_idx(p, r):
        return ((p // n_cls) * n_ch + p % n_cls + 1, 0, 0)

    colorized = pl.pallas_call(
        functools.partial(_attn_kernel, rb_size),
        grid=(n_pairs, n_rb),
        in_specs=[
            pl.BlockSpec((1, c, hw), lambda p, r: (p // n_cls, 0, 0)),
            pl.BlockSpec((1, c, hw), lambda p, r: (p // n_cls, 0, 0)),
            pl.BlockSpec((1, 1, hw), lab_idx),
            pl.BlockSpec((1, 1, hw), lab_idx),
            pl.BlockSpec((1, 3, hw), lambda p, r: (p // n_cls, 0, 0)),
        ],
        out_specs=pl.BlockSpec((1, rb_size, 3), lambda p, r: (p, r, 0)),
        out_shape=jax.ShapeDtypeStruct((n_pairs, hw, 3), jnp.float32),
        scratch_shapes=[
            pltpu.VMEM((c, hw), jnp.bfloat16),
            pltpu.VMEM((c, hw), jnp.bfloat16),
            pltpu.VMEM((4, hw), jnp.bfloat16),
        ],
    )(gf, rf, gl, rl, vi)

    canvas = pl.pallas_call(
        functools.partial(_merge_kernel, n_ch),
        grid=(b,),
        in_specs=[
            pl.BlockSpec((n_cls, hw, 3), lambda i: (i, 0, 0)),
            pl.BlockSpec((n_ch, 1, hw), lambda i: (i, 0, 0)),
            pl.BlockSpec((n_ch, 1, hw), lambda i: (i, 0, 0)),
        ],
        out_specs=pl.BlockSpec((1, 3, hw), lambda i: (i, 0, 0)),
        out_shape=jax.ShapeDtypeStruct((b, 3, hw), jnp.float32),
    )(colorized, gl, rl)
    return canvas.reshape(b, 3, h, w)
